# baseline (speedup 1.0000x reference)
import functools

import jax
import jax.numpy as jnp
from jax import lax
from jax.experimental import pallas as pl
from jax.experimental.pallas import tpu as pltpu
import jax.experimental.pallas.tpu_sc as plsc

N_NODES = 10000
NPAD = 10240
NC, NS = 2, 16
NW = NC * NS
RPT = NPAD // NS
TC_BLK = 1024

AGG_CFG = {128: (128, 2), 64: (512, 2), 32: (1024, 2), 16: (2048, 2)}
K_DEG = 1024


def _epad_for(unit, e_tot):
    return -(-e_tot // unit) * unit


@functools.lru_cache(maxsize=None)
def _sc_agg(dout, ept, kk, nbg):
    nchunk = ept // kk
    ngroup = nchunk // nbg
    mesh = plsc.VectorSubcoreMesh(core_axis_name="c", subcore_axis_name="s", num_cores=NC, num_subcores=NS)

    scratch = []
    scratch += [pltpu.VMEM((kk,), jnp.int32) for _ in range(nbg)]
    scratch += [pltpu.VMEM((kk,), jnp.int32) for _ in range(nbg)]
    scratch += [pltpu.VMEM((kk, dout), jnp.float32) for _ in range(nbg)]
    scratch += [pltpu.VMEM_SHARED((NPAD, dout), jnp.float32)]
    scratch += [pltpu.SemaphoreType.DMA for _ in range(nbg)]

    @functools.partial(
        pl.kernel,
        out_type=jax.ShapeDtypeStruct((NC, NPAD, dout), jnp.float32),
        mesh=mesh,
        scratch_types=scratch,
        compiler_params=pltpu.CompilerParams(use_tc_tiling_on_sc=False),
    )
    def agg(hs_hbm, src_hbm, dst_hbm, zrows_hbm, out_hbm, *scr):
        idx_s = scr[0:nbg]
        idx_d = scr[nbg:2 * nbg]
        rows = scr[2 * nbg:3 * nbg]
        acc = scr[3 * nbg]
        sems = scr[3 * nbg + 1:3 * nbg + 1 + nbg]

        c = lax.axis_index("c")
        s = lax.axis_index("s")
        tile = c * NS + s
        base = tile * ept

        for b in range(nbg):
            off = base + b * kk
            pltpu.sync_copy(src_hbm.at[pl.ds(off, kk)], idx_s[b])
            pltpu.sync_copy(dst_hbm.at[pl.ds(off, kk)], idx_d[b])
            pltpu.async_copy(hs_hbm.at[idx_s[b]], rows[b], sems[b])
        pltpu.sync_copy(zrows_hbm, acc.at[pl.ds(s * RPT, RPT)])
        plsc.subcore_barrier()

        def group(g, carry):
            for b in range(nbg):
                pltpu.make_async_copy(hs_hbm.at[idx_s[b]], rows[b], sems[b]).wait()
                pltpu.sync_copy(rows[b], acc.at[idx_d[b]], add=True)
                off = base + ((g + 1) * nbg + b) * kk
                pltpu.sync_copy(src_hbm.at[pl.ds(off, kk)], idx_s[b])
                pltpu.sync_copy(dst_hbm.at[pl.ds(off, kk)], idx_d[b])
                pltpu.async_copy(hs_hbm.at[idx_s[b]], rows[b], sems[b])
            return carry

        lax.fori_loop(0, ngroup - 1, group, 0)

        for b in range(nbg):
            pltpu.make_async_copy(hs_hbm.at[idx_s[b]], rows[b], sems[b]).wait()
            pltpu.sync_copy(rows[b], acc.at[idx_d[b]], add=True)

        plsc.subcore_barrier()
        pltpu.sync_copy(acc.at[pl.ds(s * RPT, RPT)], out_hbm.at[c, pl.ds(s * RPT, RPT)])

    return agg


def _sc_deg(ept):
    nchunk = ept // K_DEG
    mesh = plsc.VectorSubcoreMesh(core_axis_name="c", subcore_axis_name="s", num_cores=NC, num_subcores=NS)

    @functools.partial(
        pl.kernel,
        out_type=jax.ShapeDtypeStruct((NC, NPAD, 16), jnp.float32),
        mesh=mesh,
        scratch_types=[
            pltpu.VMEM((K_DEG,), jnp.int32),
            pltpu.VMEM((K_DEG, 16), jnp.float32),
            pltpu.VMEM_SHARED((NPAD, 16), jnp.float32),
        ],
        compiler_params=pltpu.CompilerParams(use_tc_tiling_on_sc=False),
    )
    def deg(dst_hbm, ones_hbm, zrows_hbm, out_hbm, idx_d, ones_v, acc):
        c = lax.axis_index("c")
        s = lax.axis_index("s")
        pltpu.sync_copy(zrows_hbm, acc.at[pl.ds(s * RPT, RPT)])
        pltpu.sync_copy(ones_hbm, ones_v)
        plsc.subcore_barrier()

        tile = c * NS + s
        base = tile * ept

        def chunk(k, carry):
            off = base + k * K_DEG
            pltpu.sync_copy(dst_hbm.at[pl.ds(off, K_DEG)], idx_d)
            pltpu.sync_copy(ones_v, acc.at[idx_d], add=True)
            return carry

        lax.fori_loop(0, nchunk, chunk, 0)
        plsc.subcore_barrier()
        pltpu.sync_copy(acc.at[pl.ds(s * RPT, RPT)], out_hbm.at[c, pl.ds(s * RPT, RPT)])

    return deg


def _tc_xw(x_pad, w1):
    def body(x_ref, w_ref, xw_ref):
        xw_ref[...] = jnp.dot(x_ref[...], w_ref[...],
                              preferred_element_type=jnp.float32)

    return pl.pallas_call(
        body,
        out_shape=jax.ShapeDtypeStruct((NPAD, 128), jnp.float32),
    )(x_pad, w1)


def _tc_scale(xw, d0, d1):
    def body(xw_ref, d0_ref, d1_ref, dis_ref, hs_ref):
        deg = (d0_ref[...] + d1_ref[...])[:, 0:1]
        row = lax.broadcasted_iota(jnp.int32, (NPAD, 1), 0)
        valid = (row < N_NODES).astype(jnp.float32)
        dis = valid * lax.rsqrt(jnp.maximum(deg, 1.0))
        dis_ref[...] = dis
        hs_ref[...] = xw_ref[...] * dis

    return pl.pallas_call(
        body,
        out_shape=[
            jax.ShapeDtypeStruct((NPAD, 1), jnp.float32),
            jax.ShapeDtypeStruct((NPAD, 128), jnp.float32),
        ],
    )(xw, d0, d1)


def _tc_mid(a0, a1, dis, b, w):
    dout = w.shape[1]

    def body(a0_ref, a1_ref, dis_ref, b_ref, w_ref, hs_ref):
        dis_v = dis_ref[...]
        h = jnp.maximum((a0_ref[...] + a1_ref[...]) * dis_v + b_ref[...], 0.0)
        hs_ref[...] = jnp.dot(h, w_ref[...],
                              preferred_element_type=jnp.float32) * dis_v

    return pl.pallas_call(
        body,
        out_shape=jax.ShapeDtypeStruct((NPAD, dout), jnp.float32),
    )(a0, a1, dis, b, w)


def _tc_last(a0, a1, dis, b):
    dout = a0.shape[1]

    def body(a0_ref, a1_ref, dis_ref, b_ref, out_ref):
        out_ref[...] = (a0_ref[...] + a1_ref[...]) * dis_ref[...] + b_ref[...]

    return pl.pallas_call(
        body,
        out_shape=jax.ShapeDtypeStruct((NPAD, dout), jnp.float32),
    )(a0, a1, dis, b)


def kernel(x, edge_index, W1, b1, W2, b2, W3, b3, W4, b4, W5, b5):
    n = x.shape[0]
    e = edge_index.shape[1]
    e_tot = e + n

    douts = [w.shape[1] for w in (W1, W2, W3, W4, W5)]
    cfgs = [AGG_CFG[d] for d in douts]
    epads = [_epad_for(NW * kk * nbg, e_tot) for kk, nbg in cfgs]
    epad_deg = _epad_for(NW * K_DEG, e_tot)
    epad_max = max(epads + [epad_deg])

    ei = edge_index.astype(jnp.int32)
    loop = jnp.arange(n, dtype=jnp.int32)
    pad = n + (jnp.arange(epad_max - e_tot, dtype=jnp.int32) % (NPAD - n))
    srcp = jnp.concatenate([ei[0], loop, pad])
    dstp = jnp.concatenate([ei[1], loop, pad])

    x_pad = jnp.pad(x, ((0, NPAD - n), (0, 0)))
    ones16 = jnp.ones((K_DEG, 16), jnp.float32)
    z16 = jnp.zeros((RPT, 16), jnp.float32)

    xw = _tc_xw(x_pad, W1)
    degp = _sc_deg(epad_deg // NW)(dstp, ones16, z16)
    dis, hs = _tc_scale(xw, degp[0], degp[1])

    ws = [W2, W3, W4, W5]
    bs = [b1, b2, b3, b4]
    for i in range(4):
        dout = hs.shape[1]
        kk, nbg = cfgs[i]
        zr = jnp.zeros((RPT, dout), jnp.float32)
        aggp = _sc_agg(dout, epads[i] // NW, kk, nbg)(hs, srcp, dstp, zr)
        hs = _tc_mid(aggp[0], aggp[1], dis, bs[i].reshape(1, -1), ws[i])

    dout = hs.shape[1]
    kk, nbg = cfgs[4]
    zr = jnp.zeros((RPT, dout), jnp.float32)
    aggp = _sc_agg(dout, epads[4] // NW, kk, nbg)(hs, srcp, dstp, zr)
    out = _tc_last(aggp[0], aggp[1], dis, b5.reshape(1, -1))
    return out[:n]


# speedup vs baseline: 1.0121x; 1.0121x over previous
import functools

import jax
import jax.numpy as jnp
from jax import lax
from jax.experimental import pallas as pl
from jax.experimental.pallas import tpu as pltpu
import jax.experimental.pallas.tpu_sc as plsc

N_NODES = 10000
NPAD = 10240
NC, NS = 2, 16
NW = NC * NS
RPT = NPAD // NS
TC_BLK = 1024

AGG_CFG = {128: (128, 2), 64: (512, 2), 32: (1024, 2), 16: (2048, 2)}
K_DEG = 1024


def _epad_for(unit, e_tot):
    return -(-e_tot // unit) * unit


@functools.lru_cache(maxsize=None)
def _sc_agg(dout, ept, kk, nbg):
    nchunk = ept // kk
    ngroup = nchunk // nbg
    mesh = plsc.VectorSubcoreMesh(core_axis_name="c", subcore_axis_name="s", num_cores=NC, num_subcores=NS)

    scratch = []
    scratch += [pltpu.VMEM((kk,), jnp.int32) for _ in range(nbg)]
    scratch += [pltpu.VMEM((kk,), jnp.int32) for _ in range(nbg)]
    scratch += [pltpu.VMEM((kk, dout), jnp.float32) for _ in range(nbg)]
    scratch += [pltpu.VMEM_SHARED((NPAD, dout), jnp.float32)]
    scratch += [pltpu.SemaphoreType.DMA for _ in range(nbg)]

    @functools.partial(
        pl.kernel,
        out_type=jax.ShapeDtypeStruct((NC, NPAD, dout), jnp.float32),
        mesh=mesh,
        scratch_types=scratch,
        compiler_params=pltpu.CompilerParams(use_tc_tiling_on_sc=False),
    )
    def agg(hs_hbm, src_hbm, dst_hbm, zrows_hbm, out_hbm, *scr):
        idx_s = scr[0:nbg]
        idx_d = scr[nbg:2 * nbg]
        rows = scr[2 * nbg:3 * nbg]
        acc = scr[3 * nbg]
        sems = scr[3 * nbg + 1:3 * nbg + 1 + nbg]

        c = lax.axis_index("c")
        s = lax.axis_index("s")
        tile = c * NS + s
        base = tile * ept

        for b in range(nbg):
            off = base + b * kk
            pltpu.sync_copy(src_hbm.at[pl.ds(off, kk)], idx_s[b])
            pltpu.sync_copy(dst_hbm.at[pl.ds(off, kk)], idx_d[b])
            pltpu.async_copy(hs_hbm.at[idx_s[b]], rows[b], sems[b])
        pltpu.sync_copy(zrows_hbm, acc.at[pl.ds(s * RPT, RPT)])
        plsc.subcore_barrier()

        def group(g, carry):
            for b in range(nbg):
                pltpu.make_async_copy(hs_hbm.at[idx_s[b]], rows[b], sems[b]).wait()
                pltpu.sync_copy(rows[b], acc.at[idx_d[b]], add=True)
                off = base + ((g + 1) * nbg + b) * kk
                pltpu.sync_copy(src_hbm.at[pl.ds(off, kk)], idx_s[b])
                pltpu.sync_copy(dst_hbm.at[pl.ds(off, kk)], idx_d[b])
                pltpu.async_copy(hs_hbm.at[idx_s[b]], rows[b], sems[b])
            return carry

        lax.fori_loop(0, ngroup - 1, group, 0)

        for b in range(nbg):
            pltpu.make_async_copy(hs_hbm.at[idx_s[b]], rows[b], sems[b]).wait()
            pltpu.sync_copy(rows[b], acc.at[idx_d[b]], add=True)

        plsc.subcore_barrier()
        pltpu.sync_copy(acc.at[pl.ds(s * RPT, RPT)], out_hbm.at[c, pl.ds(s * RPT, RPT)])

    return agg


def _sc_deg(ept):
    nchunk = ept // K_DEG
    mesh = plsc.VectorSubcoreMesh(core_axis_name="c", subcore_axis_name="s", num_cores=NC, num_subcores=NS)

    @functools.partial(
        pl.kernel,
        out_type=jax.ShapeDtypeStruct((NC, NPAD, 16), jnp.float32),
        mesh=mesh,
        scratch_types=[
            pltpu.VMEM((K_DEG,), jnp.int32),
            pltpu.VMEM((K_DEG,), jnp.int32),
            pltpu.VMEM((K_DEG, 16), jnp.float32),
            pltpu.VMEM_SHARED((NPAD, 16), jnp.float32),
            pltpu.SemaphoreType.DMA,
            pltpu.SemaphoreType.DMA,
        ],
        compiler_params=pltpu.CompilerParams(use_tc_tiling_on_sc=False),
    )
    def deg(dst_hbm, ones_hbm, zrows_hbm, out_hbm, idx0, idx1, ones_v, acc, sem0, sem1):
        idx = (idx0, idx1)
        sems = (sem0, sem1)
        c = lax.axis_index("c")
        s = lax.axis_index("s")
        tile = c * NS + s
        base = tile * ept

        for b in range(2):
            off = base + b * K_DEG
            pltpu.async_copy(dst_hbm.at[pl.ds(off, K_DEG)], idx[b], sems[b])
        pltpu.sync_copy(zrows_hbm, acc.at[pl.ds(s * RPT, RPT)])
        pltpu.sync_copy(ones_hbm, ones_v)
        plsc.subcore_barrier()

        def group(g, carry):
            for b in range(2):
                pltpu.make_async_copy(dst_hbm.at[pl.ds(0, K_DEG)], idx[b], sems[b]).wait()
                pltpu.sync_copy(ones_v, acc.at[idx[b]], add=True)
                off = base + ((g + 1) * 2 + b) * K_DEG
                pltpu.async_copy(dst_hbm.at[pl.ds(off, K_DEG)], idx[b], sems[b])
            return carry

        lax.fori_loop(0, nchunk // 2 - 1, group, 0)
        for b in range(2):
            pltpu.make_async_copy(dst_hbm.at[pl.ds(0, K_DEG)], idx[b], sems[b]).wait()
            pltpu.sync_copy(ones_v, acc.at[idx[b]], add=True)

        plsc.subcore_barrier()
        pltpu.sync_copy(acc.at[pl.ds(s * RPT, RPT)], out_hbm.at[c, pl.ds(s * RPT, RPT)])

    return deg


def _tc_first(x_pad, d0, d1, w1):
    def body(x_ref, d0_ref, d1_ref, w_ref, dis_ref, hs_ref):
        deg = (d0_ref[...] + d1_ref[...])[:, 0:1]
        row = lax.broadcasted_iota(jnp.int32, (NPAD, 1), 0)
        valid = (row < N_NODES).astype(jnp.float32)
        dis = valid * lax.rsqrt(jnp.maximum(deg, 1.0))
        dis_ref[...] = dis
        hs_ref[...] = jnp.dot(x_ref[...], w_ref[...],
                              preferred_element_type=jnp.float32) * dis

    return pl.pallas_call(
        body,
        out_shape=[
            jax.ShapeDtypeStruct((NPAD, 1), jnp.float32),
            jax.ShapeDtypeStruct((NPAD, 128), jnp.float32),
        ],
    )(x_pad, d0, d1, w1)


def _tc_mid(a0, a1, dis, b, w):
    dout = w.shape[1]

    def body(a0_ref, a1_ref, dis_ref, b_ref, w_ref, hs_ref):
        dis_v = dis_ref[...]
        h = jnp.maximum((a0_ref[...] + a1_ref[...]) * dis_v + b_ref[...], 0.0)
        hs_ref[...] = jnp.dot(h, w_ref[...],
                              preferred_element_type=jnp.float32) * dis_v

    return pl.pallas_call(
        body,
        out_shape=jax.ShapeDtypeStruct((NPAD, dout), jnp.float32),
    )(a0, a1, dis, b, w)


def _tc_last(a0, a1, dis, b):
    dout = a0.shape[1]

    def body(a0_ref, a1_ref, dis_ref, b_ref, out_ref):
        out_ref[...] = (a0_ref[...] + a1_ref[...]) * dis_ref[...] + b_ref[...]

    return pl.pallas_call(
        body,
        out_shape=jax.ShapeDtypeStruct((NPAD, dout), jnp.float32),
    )(a0, a1, dis, b)


def kernel(x, edge_index, W1, b1, W2, b2, W3, b3, W4, b4, W5, b5):
    n = x.shape[0]
    e = edge_index.shape[1]
    e_tot = e + n

    douts = [w.shape[1] for w in (W1, W2, W3, W4, W5)]
    cfgs = [AGG_CFG[d] for d in douts]
    epads = [_epad_for(NW * kk * nbg, e_tot) for kk, nbg in cfgs]
    epad_deg = _epad_for(NW * K_DEG * 2, e_tot)
    epad_max = max(epads + [epad_deg])

    ei = edge_index.astype(jnp.int32)
    loop = jnp.arange(n, dtype=jnp.int32)
    pad = n + (jnp.arange(epad_max - e_tot, dtype=jnp.int32) % (NPAD - n))
    srcp = jnp.concatenate([ei[0], loop, pad])
    dstp = jnp.concatenate([ei[1], loop, pad])

    x_pad = jnp.pad(x, ((0, NPAD - n), (0, 0)))
    ones16 = jnp.ones((K_DEG, 16), jnp.float32)
    z16 = jnp.zeros((RPT, 16), jnp.float32)

    degp = _sc_deg(epad_deg // NW)(dstp, ones16, z16)
    dis, hs = _tc_first(x_pad, degp[0], degp[1], W1)

    ws = [W2, W3, W4, W5]
    bs = [b1, b2, b3, b4]
    for i in range(4):
        dout = hs.shape[1]
        kk, nbg = cfgs[i]
        zr = jnp.zeros((RPT, dout), jnp.float32)
        aggp = _sc_agg(dout, epads[i] // NW, kk, nbg)(hs, srcp, dstp, zr)
        hs = _tc_mid(aggp[0], aggp[1], dis, bs[i].reshape(1, -1), ws[i])

    dout = hs.shape[1]
    kk, nbg = cfgs[4]
    zr = jnp.zeros((RPT, dout), jnp.float32)
    aggp = _sc_agg(dout, epads[4] // NW, kk, nbg)(hs, srcp, dstp, zr)
    out = _tc_last(aggp[0], aggp[1], dis, b5.reshape(1, -1))
    return out[:n]


# speedup vs baseline: 1.0431x; 1.0306x over previous
import functools

import jax
import jax.numpy as jnp
from jax import lax
from jax.experimental import pallas as pl
from jax.experimental.pallas import tpu as pltpu
import jax.experimental.pallas.tpu_sc as plsc

N_NODES = 10000
NPAD = 10240
NC, NS = 2, 16
NW = NC * NS
RPT = NPAD // NS
TC_BLK = 1024

AGG_CFG = {128: (160, 2), 64: (512, 2), 32: (1536, 2), 16: (3072, 2)}
K_DEG = 1024


def _epad_for(unit, e_tot):
    return -(-e_tot // unit) * unit


@functools.lru_cache(maxsize=None)
def _sc_agg(dout, ept, kk, nbg):
    nchunk = ept // kk
    ngroup = nchunk // nbg
    mesh = plsc.VectorSubcoreMesh(core_axis_name="c", subcore_axis_name="s", num_cores=NC, num_subcores=NS)

    scratch = []
    scratch += [pltpu.VMEM((kk,), jnp.int32) for _ in range(nbg)]
    scratch += [pltpu.VMEM((kk,), jnp.int32) for _ in range(nbg)]
    scratch += [pltpu.VMEM((kk, dout), jnp.float32) for _ in range(nbg)]
    scratch += [pltpu.VMEM_SHARED((NPAD, dout), jnp.float32)]
    scratch += [pltpu.SemaphoreType.DMA for _ in range(nbg)]

    @functools.partial(
        pl.kernel,
        out_type=jax.ShapeDtypeStruct((NC, NPAD, dout), jnp.float32),
        mesh=mesh,
        scratch_types=scratch,
        compiler_params=pltpu.CompilerParams(use_tc_tiling_on_sc=False),
    )
    def agg(hs_hbm, src_hbm, dst_hbm, zrows_hbm, out_hbm, *scr):
        idx_s = scr[0:nbg]
        idx_d = scr[nbg:2 * nbg]
        rows = scr[2 * nbg:3 * nbg]
        acc = scr[3 * nbg]
        sems = scr[3 * nbg + 1:3 * nbg + 1 + nbg]

        c = lax.axis_index("c")
        s = lax.axis_index("s")
        tile = c * NS + s
        base = tile * ept

        for b in range(nbg):
            off = base + b * kk
            pltpu.sync_copy(src_hbm.at[pl.ds(off, kk)], idx_s[b])
            pltpu.sync_copy(dst_hbm.at[pl.ds(off, kk)], idx_d[b])
            pltpu.async_copy(hs_hbm.at[idx_s[b]], rows[b], sems[b])
        pltpu.sync_copy(zrows_hbm, acc.at[pl.ds(s * RPT, RPT)])
        plsc.subcore_barrier()

        def group(g, carry):
            for b in range(nbg):
                pltpu.make_async_copy(hs_hbm.at[idx_s[b]], rows[b], sems[b]).wait()
                pltpu.sync_copy(rows[b], acc.at[idx_d[b]], add=True)
                off = base + ((g + 1) * nbg + b) * kk
                pltpu.sync_copy(src_hbm.at[pl.ds(off, kk)], idx_s[b])
                pltpu.sync_copy(dst_hbm.at[pl.ds(off, kk)], idx_d[b])
                pltpu.async_copy(hs_hbm.at[idx_s[b]], rows[b], sems[b])
            return carry

        lax.fori_loop(0, ngroup - 1, group, 0)

        for b in range(nbg):
            pltpu.make_async_copy(hs_hbm.at[idx_s[b]], rows[b], sems[b]).wait()
            pltpu.sync_copy(rows[b], acc.at[idx_d[b]], add=True)

        plsc.subcore_barrier()
        pltpu.sync_copy(acc.at[pl.ds(s * RPT, RPT)], out_hbm.at[c, pl.ds(s * RPT, RPT)])

    return agg


def _sc_deg(ept):
    nchunk = ept // K_DEG
    mesh = plsc.VectorSubcoreMesh(core_axis_name="c", subcore_axis_name="s", num_cores=NC, num_subcores=NS)

    @functools.partial(
        pl.kernel,
        out_type=jax.ShapeDtypeStruct((NC, NPAD, 16), jnp.float32),
        mesh=mesh,
        scratch_types=[
            pltpu.VMEM((K_DEG,), jnp.int32),
            pltpu.VMEM((K_DEG,), jnp.int32),
            pltpu.VMEM((K_DEG, 16), jnp.float32),
            pltpu.VMEM_SHARED((NPAD, 16), jnp.float32),
            pltpu.SemaphoreType.DMA,
            pltpu.SemaphoreType.DMA,
        ],
        compiler_params=pltpu.CompilerParams(use_tc_tiling_on_sc=False),
    )
    def deg(dst_hbm, ones_hbm, zrows_hbm, out_hbm, idx0, idx1, ones_v, acc, sem0, sem1):
        idx = (idx0, idx1)
        sems = (sem0, sem1)
        c = lax.axis_index("c")
        s = lax.axis_index("s")
        tile = c * NS + s
        base = tile * ept

        for b in range(2):
            off = base + b * K_DEG
            pltpu.async_copy(dst_hbm.at[pl.ds(off, K_DEG)], idx[b], sems[b])
        pltpu.sync_copy(zrows_hbm, acc.at[pl.ds(s * RPT, RPT)])
        pltpu.sync_copy(ones_hbm, ones_v)
        plsc.subcore_barrier()

        def group(g, carry):
            for b in range(2):
                pltpu.make_async_copy(dst_hbm.at[pl.ds(0, K_DEG)], idx[b], sems[b]).wait()
                pltpu.sync_copy(ones_v, acc.at[idx[b]], add=True)
                off = base + ((g + 1) * 2 + b) * K_DEG
                pltpu.async_copy(dst_hbm.at[pl.ds(off, K_DEG)], idx[b], sems[b])
            return carry

        lax.fori_loop(0, nchunk // 2 - 1, group, 0)
        for b in range(2):
            pltpu.make_async_copy(dst_hbm.at[pl.ds(0, K_DEG)], idx[b], sems[b]).wait()
            pltpu.sync_copy(ones_v, acc.at[idx[b]], add=True)

        plsc.subcore_barrier()
        pltpu.sync_copy(acc.at[pl.ds(s * RPT, RPT)], out_hbm.at[c, pl.ds(s * RPT, RPT)])

    return deg


def _tc_first(x_pad, d0, d1, w1):
    def body(x_ref, d0_ref, d1_ref, w_ref, dis_ref, hs_ref):
        deg = (d0_ref[...] + d1_ref[...])[:, 0:1]
        row = lax.broadcasted_iota(jnp.int32, (NPAD, 1), 0)
        valid = (row < N_NODES).astype(jnp.float32)
        dis = valid * lax.rsqrt(jnp.maximum(deg, 1.0))
        dis_ref[...] = dis
        hs_ref[...] = jnp.dot(x_ref[...], w_ref[...],
                              preferred_element_type=jnp.float32) * dis

    return pl.pallas_call(
        body,
        out_shape=[
            jax.ShapeDtypeStruct((NPAD, 1), jnp.float32),
            jax.ShapeDtypeStruct((NPAD, 128), jnp.float32),
        ],
    )(x_pad, d0, d1, w1)


def _tc_mid(a0, a1, dis, b, w):
    dout = w.shape[1]

    def body(a0_ref, a1_ref, dis_ref, b_ref, w_ref, hs_ref):
        dis_v = dis_ref[...]
        h = jnp.maximum((a0_ref[...] + a1_ref[...]) * dis_v + b_ref[...], 0.0)
        hs_ref[...] = jnp.dot(h, w_ref[...],
                              preferred_element_type=jnp.float32) * dis_v

    return pl.pallas_call(
        body,
        out_shape=jax.ShapeDtypeStruct((NPAD, dout), jnp.float32),
    )(a0, a1, dis, b, w)


def _tc_last(a0, a1, dis, b):
    dout = a0.shape[1]

    def body(a0_ref, a1_ref, dis_ref, b_ref, out_ref):
        out_ref[...] = (a0_ref[...] + a1_ref[...]) * dis_ref[...] + b_ref[...]

    return pl.pallas_call(
        body,
        out_shape=jax.ShapeDtypeStruct((NPAD, dout), jnp.float32),
    )(a0, a1, dis, b)


def kernel(x, edge_index, W1, b1, W2, b2, W3, b3, W4, b4, W5, b5):
    n = x.shape[0]
    e = edge_index.shape[1]
    e_tot = e + n

    douts = [w.shape[1] for w in (W1, W2, W3, W4, W5)]
    cfgs = [AGG_CFG[d] for d in douts]
    epads = [_epad_for(NW * kk * nbg, e_tot) for kk, nbg in cfgs]
    epad_deg = _epad_for(NW * K_DEG * 2, e_tot)
    epad_max = max(epads + [epad_deg])

    ei = edge_index.astype(jnp.int32)
    loop = jnp.arange(n, dtype=jnp.int32)
    pad = n + (jnp.arange(epad_max - e_tot, dtype=jnp.int32) % (NPAD - n))
    srcp = jnp.concatenate([ei[0], loop, pad])
    dstp = jnp.concatenate([ei[1], loop, pad])

    x_pad = jnp.pad(x, ((0, NPAD - n), (0, 0)))
    ones16 = jnp.ones((K_DEG, 16), jnp.float32)
    z16 = jnp.zeros((RPT, 16), jnp.float32)

    degp = _sc_deg(epad_deg // NW)(dstp, ones16, z16)
    dis, hs = _tc_first(x_pad, degp[0], degp[1], W1)

    ws = [W2, W3, W4, W5]
    bs = [b1, b2, b3, b4]
    for i in range(4):
        dout = hs.shape[1]
        kk, nbg = cfgs[i]
        zr = jnp.zeros((RPT, dout), jnp.float32)
        aggp = _sc_agg(dout, epads[i] // NW, kk, nbg)(hs, srcp, dstp, zr)
        hs = _tc_mid(aggp[0], aggp[1], dis, bs[i].reshape(1, -1), ws[i])

    dout = hs.shape[1]
    kk, nbg = cfgs[4]
    zr = jnp.zeros((RPT, dout), jnp.float32)
    aggp = _sc_agg(dout, epads[4] // NW, kk, nbg)(hs, srcp, dstp, zr)
    out = _tc_last(aggp[0], aggp[1], dis, b5.reshape(1, -1))
    return out[:n]


# speedup vs baseline: 1.0432x; 1.0002x over previous
import functools

import jax
import jax.numpy as jnp
from jax import lax
from jax.experimental import pallas as pl
from jax.experimental.pallas import tpu as pltpu
import jax.experimental.pallas.tpu_sc as plsc

N_NODES = 10000
NPAD = 10240
NC, NS = 2, 16
NW = NC * NS
RPT = NPAD // NS

AGG_CFG = {128: (160, 2), 64: (512, 2), 32: (1536, 2), 16: (3072, 2)}
K_DEG = 1024


def _epad_for(unit, e_tot):
    return -(-e_tot // unit) * unit


@functools.lru_cache(maxsize=None)
def _sc_agg(dout, ept, kk, nbg):
    nchunk = ept // kk
    ngroup = nchunk // nbg
    mesh = plsc.VectorSubcoreMesh(core_axis_name="c", subcore_axis_name="s", num_cores=NC, num_subcores=NS)

    scratch = []
    scratch += [pltpu.VMEM((kk,), jnp.int32) for _ in range(nbg)]
    scratch += [pltpu.VMEM((kk,), jnp.int32) for _ in range(nbg)]
    scratch += [pltpu.VMEM((kk, dout), jnp.float32) for _ in range(nbg)]
    scratch += [pltpu.VMEM_SHARED((NPAD, dout), jnp.float32)]
    scratch += [pltpu.SemaphoreType.DMA for _ in range(nbg)]

    @functools.partial(
        pl.kernel,
        out_type=jax.ShapeDtypeStruct((NC, NPAD, dout), jnp.float32),
        mesh=mesh,
        scratch_types=scratch,
        compiler_params=pltpu.CompilerParams(use_tc_tiling_on_sc=False),
    )
    def agg(hs_hbm, src_hbm, dst_hbm, zrows_hbm, out_hbm, *scr):
        idx_s = scr[0:nbg]
        idx_d = scr[nbg:2 * nbg]
        rows = scr[2 * nbg:3 * nbg]
        acc = scr[3 * nbg]
        sems = scr[3 * nbg + 1:3 * nbg + 1 + nbg]

        c = lax.axis_index("c")
        s = lax.axis_index("s")
        tile = c * NS + s
        base = tile * ept

        for b in range(nbg):
            off = base + b * kk
            pltpu.sync_copy(src_hbm.at[pl.ds(off, kk)], idx_s[b])
            pltpu.sync_copy(dst_hbm.at[pl.ds(off, kk)], idx_d[b])
            pltpu.async_copy(hs_hbm.at[idx_s[b]], rows[b], sems[b])
        pltpu.sync_copy(zrows_hbm, acc.at[pl.ds(s * RPT, RPT)])
        plsc.subcore_barrier()

        def group(g, carry):
            for b in range(nbg):
                pltpu.make_async_copy(hs_hbm.at[idx_s[b]], rows[b], sems[b]).wait()
                pltpu.sync_copy(rows[b], acc.at[idx_d[b]], add=True)
                off = base + ((g + 1) * nbg + b) * kk
                pltpu.sync_copy(src_hbm.at[pl.ds(off, kk)], idx_s[b])
                pltpu.sync_copy(dst_hbm.at[pl.ds(off, kk)], idx_d[b])
                pltpu.async_copy(hs_hbm.at[idx_s[b]], rows[b], sems[b])
            return carry

        lax.fori_loop(0, ngroup - 1, group, 0)

        for b in range(nbg):
            pltpu.make_async_copy(hs_hbm.at[idx_s[b]], rows[b], sems[b]).wait()
            pltpu.sync_copy(rows[b], acc.at[idx_d[b]], add=True)

        plsc.subcore_barrier()
        pltpu.sync_copy(acc.at[pl.ds(s * RPT, RPT)], out_hbm.at[c, pl.ds(s * RPT, RPT)])

    return agg


def _sc_deg(ept):
    nchunk = ept // K_DEG
    mesh = plsc.VectorSubcoreMesh(core_axis_name="c", subcore_axis_name="s", num_cores=NC, num_subcores=NS)

    @functools.partial(
        pl.kernel,
        out_type=jax.ShapeDtypeStruct((NC, NPAD, 16), jnp.float32),
        mesh=mesh,
        scratch_types=[
            pltpu.VMEM((K_DEG,), jnp.int32),
            pltpu.VMEM((K_DEG,), jnp.int32),
            pltpu.VMEM((K_DEG, 16), jnp.float32),
            pltpu.VMEM_SHARED((NPAD, 16), jnp.float32),
            pltpu.SemaphoreType.DMA,
            pltpu.SemaphoreType.DMA,
        ],
        compiler_params=pltpu.CompilerParams(use_tc_tiling_on_sc=False),
    )
    def deg(dst_hbm, ones_hbm, zrows_hbm, out_hbm, idx0, idx1, ones_v, acc, sem0, sem1):
        idx = (idx0, idx1)
        sems = (sem0, sem1)
        c = lax.axis_index("c")
        s = lax.axis_index("s")
        tile = c * NS + s
        base = tile * ept

        for b in range(2):
            off = base + b * K_DEG
            pltpu.async_copy(dst_hbm.at[pl.ds(off, K_DEG)], idx[b], sems[b])
        pltpu.sync_copy(zrows_hbm, acc.at[pl.ds(s * RPT, RPT)])
        pltpu.sync_copy(ones_hbm, ones_v)
        plsc.subcore_barrier()

        def group(g, carry):
            for b in range(2):
                pltpu.make_async_copy(dst_hbm.at[pl.ds(0, K_DEG)], idx[b], sems[b]).wait()
                pltpu.sync_copy(ones_v, acc.at[idx[b]], add=True)
                off = base + ((g + 1) * 2 + b) * K_DEG
                pltpu.async_copy(dst_hbm.at[pl.ds(off, K_DEG)], idx[b], sems[b])
            return carry

        lax.fori_loop(0, nchunk // 2 - 1, group, 0)
        for b in range(2):
            pltpu.make_async_copy(dst_hbm.at[pl.ds(0, K_DEG)], idx[b], sems[b]).wait()
            pltpu.sync_copy(ones_v, acc.at[idx[b]], add=True)

        plsc.subcore_barrier()
        pltpu.sync_copy(acc.at[pl.ds(s * RPT, RPT)], out_hbm.at[c, pl.ds(s * RPT, RPT)])

    return deg


def _tc_first(x_pad, d0, d1, w1):
    def body(x_ref, d0_ref, d1_ref, w_ref, dis_ref, hs_ref):
        deg = (d0_ref[...] + d1_ref[...])[:, 0:1]
        row = lax.broadcasted_iota(jnp.int32, (NPAD, 1), 0)
        valid = (row < N_NODES).astype(jnp.float32)
        dis = valid * lax.rsqrt(jnp.maximum(deg, 1.0))
        dis_ref[...] = dis
        hs_ref[...] = jnp.dot(x_ref[...], w_ref[...],
                              preferred_element_type=jnp.float32) * dis

    return pl.pallas_call(
        body,
        out_shape=[
            jax.ShapeDtypeStruct((NPAD, 1), jnp.float32),
            jax.ShapeDtypeStruct((NPAD, 128), jnp.float32),
        ],
    )(x_pad, d0, d1, w1)


def _tc_mid(a0, a1, dis, b, w):
    dout = w.shape[1]

    def body(a0_ref, a1_ref, dis_ref, b_ref, w_ref, hs_ref):
        dis_v = dis_ref[...]
        h = jnp.maximum((a0_ref[...] + a1_ref[...]) * dis_v + b_ref[...], 0.0)
        hs_ref[...] = jnp.dot(h, w_ref[...],
                              preferred_element_type=jnp.float32) * dis_v

    return pl.pallas_call(
        body,
        out_shape=jax.ShapeDtypeStruct((NPAD, dout), jnp.float32),
    )(a0, a1, dis, b, w)


def _tc_last(a0, a1, dis, b):
    dout = a0.shape[1]

    def body(a0_ref, a1_ref, dis_ref, b_ref, out_ref):
        out_ref[...] = (a0_ref[...] + a1_ref[...]) * dis_ref[...] + b_ref[...]

    return pl.pallas_call(
        body,
        out_shape=jax.ShapeDtypeStruct((NPAD, dout), jnp.float32),
    )(a0, a1, dis, b)


def kernel(x, edge_index, W1, b1, W2, b2, W3, b3, W4, b4, W5, b5):
    n = x.shape[0]
    e = edge_index.shape[1]
    e_tot = e + n

    douts = [w.shape[1] for w in (W1, W2, W3, W4, W5)]
    cfgs = [AGG_CFG[d] for d in douts]
    epads = [_epad_for(NW * kk * nbg, e_tot) for kk, nbg in cfgs]
    epad_deg = _epad_for(NW * K_DEG * 2, e_tot)
    epad_max = max(epads + [epad_deg])

    ei = edge_index.astype(jnp.int32)
    loop = jnp.arange(n, dtype=jnp.int32)
    pad = n + (jnp.arange(epad_max - e_tot, dtype=jnp.int32) % (NPAD - n))
    srcp = jnp.concatenate([ei[0], loop, pad])
    dstp = jnp.concatenate([ei[1], loop, pad])

    x_pad = jnp.pad(x, ((0, NPAD - n), (0, 0)))
    ones16 = jnp.ones((K_DEG, 16), jnp.float32)
    z16 = jnp.zeros((RPT, 16), jnp.float32)

    degp = _sc_deg(epad_deg // NW)(dstp, ones16, z16)
    dis, hs = _tc_first(x_pad, degp[0], degp[1], W1)

    ws = [W2, W3, W4, W5]
    bs = [b1, b2, b3, b4]
    for i in range(4):
        dout = hs.shape[1]
        kk, nbg = cfgs[i]
        zr = jnp.zeros((RPT, dout), jnp.float32)
        aggp = _sc_agg(dout, epads[i] // NW, kk, nbg)(hs, srcp, dstp, zr)
        hs = _tc_mid(aggp[0], aggp[1], dis, bs[i].reshape(1, -1), ws[i])

    dout = hs.shape[1]
    kk, nbg = cfgs[4]
    zr = jnp.zeros((RPT, dout), jnp.float32)
    aggp = _sc_agg(dout, epads[4] // NW, kk, nbg)(hs, srcp, dstp, zr)
    out = _tc_last(aggp[0], aggp[1], dis, b5.reshape(1, -1))
    return out[:n]
